# Initial kernel scaffold; baseline (speedup 1.0000x reference)
import jax
import jax.numpy as jnp
from jax.experimental import pallas as pl


def kernel(h_A, herb_idx_A, h_B, herb_idx_B, W1, b1, W2, b2, F1, fb1, F2, fb2):
    raise NotImplementedError("write your pallas kernel here")


# speedup vs baseline: 4.3075x; 4.3075x over previous
import dataclasses
import functools

import jax
import jax.numpy as jnp
from jax import lax
from jax.experimental import pallas as pl
from jax.experimental.pallas import tpu as pltpu
from jax.experimental.pallas import tpu_sc as plsc

NSEG = 10000
NNODE = 320000
DIM = 128

NCORE = 2
NSUB = 16
RPB = 128
NBLK = NNODE // RPB
NSEGP = 10240
HSEG = NSEGP // 2
HTAB = HSEG + 8
RPT = HSEG // NSUB

_mesh = plsc.VectorSubcoreMesh(core_axis_name="c", subcore_axis_name="s")

_sc_params = pltpu.CompilerParams()
if "needs_layout_passes" in pltpu.CompilerParams.__dataclass_fields__:
    _sc_params = dataclasses.replace(_sc_params, needs_layout_passes=False)


def _f32(shape):
    return jax.ShapeDtypeStruct(shape, jnp.float32)


@functools.partial(
    pl.kernel,
    out_type=_f32((NCORE, NBLK, RPB, DIM)),
    mesh=_mesh,
    scratch_types=[
        pltpu.VMEM((RPB, DIM), jnp.float32),
        pltpu.VMEM((1, RPB), jnp.int32),
        pltpu.SemaphoreType.DMA,
    ],
)
def _sc_expand(v_stack, idx_stack, e_out, rbuf, ibuf, sem):
    c = lax.axis_index("c")
    s = lax.axis_index("s")

    @pl.loop(s, NBLK, step=NSUB)
    def _(b):
        pltpu.sync_copy(idx_stack.at[c].at[b], ibuf)
        pltpu.async_copy(v_stack.at[c].at[ibuf.at[0]], rbuf, sem).wait()
        pltpu.sync_copy(rbuf, e_out.at[c].at[b])


RB = 1000
NRB = NNODE // RB
WSEG = 128


def _one_hot_win(idxv, wbase):
    rows = wbase + lax.broadcasted_iota(jnp.int32, (WSEG, RB), 0)
    return (rows == idxv[None, :]).astype(jnp.float32)


def _tc_segcnt_body(x_ref, i_ref, tab_ref, cnt_ref):
    @pl.when(pl.program_id(0) == 0)
    def _():
        tab_ref[...] = jnp.zeros_like(tab_ref)
        cnt_ref[...] = jnp.zeros_like(cnt_ref)

    idxv = i_ref[0, 0, :]
    base = (jnp.min(idxv) // 8) * 8
    nwin = (jnp.max(idxv) - base) // WSEG + 1

    def wb(k, carry):
        wbase = base + k * WSEG
        oh = _one_hot_win(idxv, wbase)
        tab_ref[pl.ds(wbase, WSEG), :] += jnp.dot(
            oh, x_ref[...], preferred_element_type=jnp.float32)
        cnt_ref[pl.ds(wbase, WSEG), :] += jnp.broadcast_to(
            jnp.sum(oh, axis=1, keepdims=True), (WSEG, 16))
        return carry

    lax.fori_loop(0, nwin, wb, 0)


def _tc_segcnt(x, idx3):
    return pl.pallas_call(
        _tc_segcnt_body,
        grid=(NRB,),
        in_specs=[
            pl.BlockSpec((RB, DIM), lambda i: (i, 0)),
            pl.BlockSpec((1, 1, RB), lambda i: (i, 0, 0)),
        ],
        out_specs=[
            pl.BlockSpec((NSEGP, DIM), lambda i: (0, 0)),
            pl.BlockSpec((NSEGP, 16), lambda i: (0, 0)),
        ],
        out_shape=[_f32((NSEGP, DIM)), _f32((NSEGP, 16))],
    )(x, idx3)


def _tc_segpay_body(x_ref, p_ref, i_ref, tab_ref, pay_ref):
    @pl.when(pl.program_id(0) == 0)
    def _():
        tab_ref[...] = jnp.zeros_like(tab_ref)
        pay_ref[...] = jnp.zeros_like(pay_ref)

    idxv = i_ref[0, 0, :]
    base = (jnp.min(idxv) // 8) * 8
    nwin = (jnp.max(idxv) - base) // WSEG + 1

    def wb(k, carry):
        wbase = base + k * WSEG
        oh = _one_hot_win(idxv, wbase)
        tab_ref[pl.ds(wbase, WSEG), :] += jnp.dot(
            oh, x_ref[...], preferred_element_type=jnp.float32)
        pay_ref[pl.ds(wbase, WSEG), :] += jnp.dot(
            oh, p_ref[...], preferred_element_type=jnp.float32)
        return carry

    lax.fori_loop(0, nwin, wb, 0)


def _tc_segpay(x, p, idx3):
    return pl.pallas_call(
        _tc_segpay_body,
        grid=(NRB,),
        in_specs=[
            pl.BlockSpec((RB, DIM), lambda i: (i, 0)),
            pl.BlockSpec((RB, 16), lambda i: (i, 0)),
            pl.BlockSpec((1, 1, RB), lambda i: (i, 0, 0)),
        ],
        out_specs=[
            pl.BlockSpec((NSEGP, DIM), lambda i: (0, 0)),
            pl.BlockSpec((NSEGP, 16), lambda i: (0, 0)),
        ],
        out_shape=[_f32((NSEGP, DIM)), _f32((NSEGP, 16))],
    )(x, p, idx3)


def _tc_vtab_body(s_ref, c_ref, w1g_ref, b1_ref, v_ref):
    cnt = jnp.maximum(c_ref[:, 0:1], 1.0)
    mean = s_ref[...] / cnt
    v = jnp.dot(mean, w1g_ref[...],
                preferred_element_type=jnp.float32) + b1_ref[...]
    v_ref[...] = jnp.concatenate([v, jnp.zeros_like(v)], axis=1)


def _tc_vtab(s_tab, c_tab, w1g, b1r):
    blk = 1024
    return pl.pallas_call(
        _tc_vtab_body,
        grid=(NSEGP // blk,),
        in_specs=[
            pl.BlockSpec((blk, DIM), lambda i: (i, 0)),
            pl.BlockSpec((blk, 16), lambda i: (i, 0)),
            pl.BlockSpec((DIM, 64), lambda i: (0, 0)),
            pl.BlockSpec((1, 64), lambda i: (0, 0)),
        ],
        out_specs=pl.BlockSpec((blk, DIM), lambda i: (i, 0)),
        out_shape=_f32((NSEGP, DIM)),
    )(s_tab, c_tab, w1g, b1r)


def _tc_score_body(h_ref, e_in_ref, w1h_ref, w2_ref, b2_ref, e_ref, eh_ref):
    u = jnp.dot(h_ref[...], w1h_ref[...],
                preferred_element_type=jnp.float32) + e_in_ref[:, :64]
    u = jnp.where(u > 0, u, 0.2 * u)
    s = jnp.sum(u * w2_ref[...], axis=1, keepdims=True) + b2_ref[...]
    e = jnp.exp(s)
    e_ref[...] = jnp.concatenate(
        [e, jnp.zeros((e.shape[0], 15), jnp.float32)], axis=1)
    eh_ref[...] = h_ref[...] * e


def _tc_score(h, e_in, w1h, w2r, b2r):
    blk = 4000
    return pl.pallas_call(
        _tc_score_body,
        grid=(NNODE // blk,),
        in_specs=[
            pl.BlockSpec((blk, DIM), lambda i: (i, 0)),
            pl.BlockSpec((blk, DIM), lambda i: (i, 0)),
            pl.BlockSpec((DIM, 64), lambda i: (0, 0)),
            pl.BlockSpec((1, 64), lambda i: (0, 0)),
            pl.BlockSpec((1, 1), lambda i: (0, 0)),
        ],
        out_specs=[
            pl.BlockSpec((blk, 16), lambda i: (i, 0)),
            pl.BlockSpec((blk, DIM), lambda i: (i, 0)),
        ],
        out_shape=[_f32((NNODE, 16)), _f32((NNODE, DIM))],
    )(h, e_in, w1h, w2r, b2r)


def _tc_fuse_body(wsa_ref, za_ref, wsb_ref, zb_ref, f1_ref, fb1_ref, f2_ref,
                  fb2_ref, hi_ref, ha_ref, hb_ref):
    za = za_ref[:, 0:1]
    zb = zb_ref[:, 0:1]
    ha = wsa_ref[...] / (za + 1e-16)
    hb = wsb_ref[...] / (zb + 1e-16)
    ha_ref[...] = ha
    hb_ref[...] = hb
    cat = jnp.concatenate([ha, hb, ha * hb, jnp.abs(ha - hb)], axis=1)
    hid = jnp.maximum(jnp.dot(cat, f1_ref[...],
                              preferred_element_type=jnp.float32)
                      + fb1_ref[...], 0.0)
    hi_ref[...] = jnp.dot(hid, f2_ref[...],
                          preferred_element_type=jnp.float32) + fb2_ref[...]


def _tc_fuse(ws_a, z_a, ws_b, z_b, f1, fb1r, f2, fb2r):
    blk = 1000
    return pl.pallas_call(
        _tc_fuse_body,
        grid=(NSEG // blk,),
        in_specs=[
            pl.BlockSpec((blk, DIM), lambda i: (i, 0)),
            pl.BlockSpec((blk, 16), lambda i: (i, 0)),
            pl.BlockSpec((blk, DIM), lambda i: (i, 0)),
            pl.BlockSpec((blk, 16), lambda i: (i, 0)),
            pl.BlockSpec((4 * DIM, 2 * DIM), lambda i: (0, 0)),
            pl.BlockSpec((1, 2 * DIM), lambda i: (0, 0)),
            pl.BlockSpec((2 * DIM, DIM), lambda i: (0, 0)),
            pl.BlockSpec((1, DIM), lambda i: (0, 0)),
        ],
        out_specs=[
            pl.BlockSpec((blk, DIM), lambda i: (i, 0)),
            pl.BlockSpec((blk, DIM), lambda i: (i, 0)),
            pl.BlockSpec((blk, DIM), lambda i: (i, 0)),
        ],
        out_shape=[_f32((NSEG, DIM)), _f32((NSEG, DIM)), _f32((NSEG, DIM))],
    )(ws_a, z_a, ws_b, z_b, f1, fb1r, f2, fb2r)


def kernel(h_A, herb_idx_A, h_B, herb_idx_B, W1, b1, W2, b2, F1, fb1, F2, fb2):
    idx_a3 = herb_idx_A.reshape(NBLK, 1, RPB)
    idx_b3 = herb_idx_B.reshape(NBLK, 1, RPB)
    idx_ar = herb_idx_A.reshape(NRB, 1, RB)
    idx_br = herb_idx_B.reshape(NRB, 1, RB)

    s_a, c_a = _tc_segcnt(h_A, idx_ar)
    s_b, c_b = _tc_segcnt(h_B, idx_br)

    w1h = W1[:DIM]
    w1g = W1[DIM:]
    b1r = b1.reshape(1, 64)
    v_a = _tc_vtab(s_a, c_a, w1g, b1r)
    v_b = _tc_vtab(s_b, c_b, w1g, b1r)

    v_stack = jnp.stack([v_b, v_a])
    idx_stack = jnp.stack([idx_a3, idx_b3])
    e_exp = _sc_expand(v_stack, idx_stack)
    e_in_a = e_exp[0].reshape(NNODE, DIM)
    e_in_b = e_exp[1].reshape(NNODE, DIM)

    w2r = W2.reshape(1, 64)
    b2r = b2.reshape(1, 1)
    e_a, eh_a = _tc_score(h_A, e_in_a, w1h, w2r, b2r)
    e_b, eh_b = _tc_score(h_B, e_in_b, w1h, w2r, b2r)

    ws_a, z_a = _tc_segpay(eh_a, e_a, idx_ar)
    ws_b, z_b = _tc_segpay(eh_b, e_b, idx_br)

    h_int, h_agb, h_bga = _tc_fuse(ws_a, z_a, ws_b, z_b, F1,
                                   fb1.reshape(1, 2 * DIM), F2,
                                   fb2.reshape(1, DIM))
    return (h_int, h_agb, h_bga)


# speedup vs baseline: 4.8365x; 1.1228x over previous
import dataclasses
import functools

import jax
import jax.numpy as jnp
from jax import lax
from jax.experimental import pallas as pl
from jax.experimental.pallas import tpu as pltpu
from jax.experimental.pallas import tpu_sc as plsc

NSEG = 10000
NNODE = 320000
DIM = 128

NCORE = 2
NSUB = 16
RPB = 128
NBLK = NNODE // RPB
NSEGP = 10240
HSEG = NSEGP // 2
HTAB = HSEG + 8
RPT = HSEG // NSUB

_mesh = plsc.VectorSubcoreMesh(core_axis_name="c", subcore_axis_name="s")

_sc_params = pltpu.CompilerParams()
if "needs_layout_passes" in pltpu.CompilerParams.__dataclass_fields__:
    _sc_params = dataclasses.replace(_sc_params, needs_layout_passes=False)


def _f32(shape):
    return jax.ShapeDtypeStruct(shape, jnp.float32)


NQ = 4
NBLKE = NNODE // (NQ * RPB)


@functools.partial(
    pl.kernel,
    out_type=_f32((NCORE, NBLKE, NQ, RPB, DIM)),
    mesh=_mesh,
    scratch_types=[
        pltpu.VMEM((NQ * RPB, DIM), jnp.float32),
        pltpu.VMEM((NQ, RPB), jnp.int32),
        pltpu.SemaphoreType.DMA,
    ],
)
def _sc_expand(v_stack, idx_stack, e_out, rbuf, ibuf, sem):
    c = lax.axis_index("c")
    s = lax.axis_index("s")

    @pl.loop(s, NBLKE, step=NSUB)
    def _(b):
        pltpu.sync_copy(idx_stack.at[c].at[b], ibuf)
        gs = [pltpu.async_copy(v_stack.at[c].at[ibuf.at[q]],
                               rbuf.at[pl.ds(q * RPB, RPB)], sem)
              for q in range(NQ)]
        for g in gs:
            g.wait()
        ws = [pltpu.async_copy(rbuf.at[pl.ds(q * RPB, RPB)],
                               e_out.at[c].at[b].at[q], sem)
              for q in range(NQ)]
        for w in ws:
            w.wait()


RB = 1000
NRB = NNODE // RB
WSEG = 128


def _one_hot_win(idxv, wbase):
    rows = wbase + lax.broadcasted_iota(jnp.int32, (WSEG, RB), 0)
    return (rows == idxv[None, :]).astype(jnp.float32)


def _tc_segcnt_body(x_ref, i_ref, tab_ref, cnt_ref):
    @pl.when(pl.program_id(0) == 0)
    def _():
        tab_ref[...] = jnp.zeros_like(tab_ref)
        cnt_ref[...] = jnp.zeros_like(cnt_ref)

    idxv = i_ref[0, 0, :]
    base = (jnp.min(idxv) // 8) * 8
    nwin = (jnp.max(idxv) - base) // WSEG + 1

    def wb(k, carry):
        wbase = base + k * WSEG
        oh = _one_hot_win(idxv, wbase)
        tab_ref[pl.ds(wbase, WSEG), :] += jnp.dot(
            oh, x_ref[...], preferred_element_type=jnp.float32)
        cnt_ref[pl.ds(wbase, WSEG), :] += jnp.broadcast_to(
            jnp.sum(oh, axis=1, keepdims=True), (WSEG, 16))
        return carry

    lax.fori_loop(0, nwin, wb, 0)


def _tc_segcnt(x, idx3):
    return pl.pallas_call(
        _tc_segcnt_body,
        grid=(NRB,),
        in_specs=[
            pl.BlockSpec((RB, DIM), lambda i: (i, 0)),
            pl.BlockSpec((1, 1, RB), lambda i: (i, 0, 0)),
        ],
        out_specs=[
            pl.BlockSpec((NSEGP, DIM), lambda i: (0, 0)),
            pl.BlockSpec((NSEGP, 16), lambda i: (0, 0)),
        ],
        out_shape=[_f32((NSEGP, DIM)), _f32((NSEGP, 16))],
    )(x, idx3)


def _tc_segpay_body(x_ref, p_ref, i_ref, tab_ref, pay_ref):
    @pl.when(pl.program_id(0) == 0)
    def _():
        tab_ref[...] = jnp.zeros_like(tab_ref)
        pay_ref[...] = jnp.zeros_like(pay_ref)

    idxv = i_ref[0, 0, :]
    base = (jnp.min(idxv) // 8) * 8
    nwin = (jnp.max(idxv) - base) // WSEG + 1

    def wb(k, carry):
        wbase = base + k * WSEG
        oh = _one_hot_win(idxv, wbase)
        tab_ref[pl.ds(wbase, WSEG), :] += jnp.dot(
            oh, x_ref[...], preferred_element_type=jnp.float32)
        pay_ref[pl.ds(wbase, WSEG), :] += jnp.dot(
            oh, p_ref[...], preferred_element_type=jnp.float32)
        return carry

    lax.fori_loop(0, nwin, wb, 0)


def _tc_segpay(x, p, idx3):
    return pl.pallas_call(
        _tc_segpay_body,
        grid=(NRB,),
        in_specs=[
            pl.BlockSpec((RB, DIM), lambda i: (i, 0)),
            pl.BlockSpec((RB, 16), lambda i: (i, 0)),
            pl.BlockSpec((1, 1, RB), lambda i: (i, 0, 0)),
        ],
        out_specs=[
            pl.BlockSpec((NSEGP, DIM), lambda i: (0, 0)),
            pl.BlockSpec((NSEGP, 16), lambda i: (0, 0)),
        ],
        out_shape=[_f32((NSEGP, DIM)), _f32((NSEGP, 16))],
    )(x, p, idx3)


def _tc_vtab_body(s_ref, c_ref, w1g_ref, b1_ref, v_ref):
    cnt = jnp.maximum(c_ref[:, 0:1], 1.0)
    mean = s_ref[...] / cnt
    v = jnp.dot(mean, w1g_ref[...],
                preferred_element_type=jnp.float32) + b1_ref[...]
    v_ref[...] = jnp.concatenate([v, jnp.zeros_like(v)], axis=1)


def _tc_vtab(s_tab, c_tab, w1g, b1r):
    blk = 1024
    return pl.pallas_call(
        _tc_vtab_body,
        grid=(NSEGP // blk,),
        in_specs=[
            pl.BlockSpec((blk, DIM), lambda i: (i, 0)),
            pl.BlockSpec((blk, 16), lambda i: (i, 0)),
            pl.BlockSpec((DIM, 64), lambda i: (0, 0)),
            pl.BlockSpec((1, 64), lambda i: (0, 0)),
        ],
        out_specs=pl.BlockSpec((blk, DIM), lambda i: (i, 0)),
        out_shape=_f32((NSEGP, DIM)),
    )(s_tab, c_tab, w1g, b1r)


def _tc_score_body(h_ref, e_in_ref, w1h_ref, w2_ref, b2_ref, e_ref, eh_ref):
    u = jnp.dot(h_ref[...], w1h_ref[...],
                preferred_element_type=jnp.float32) + e_in_ref[:, :64]
    u = jnp.where(u > 0, u, 0.2 * u)
    s = jnp.sum(u * w2_ref[...], axis=1, keepdims=True) + b2_ref[...]
    e = jnp.exp(s)
    e_ref[...] = jnp.concatenate(
        [e, jnp.zeros((e.shape[0], 15), jnp.float32)], axis=1)
    eh_ref[...] = h_ref[...] * e


def _tc_score(h, e_in, w1h, w2r, b2r):
    blk = 4000
    return pl.pallas_call(
        _tc_score_body,
        grid=(NNODE // blk,),
        in_specs=[
            pl.BlockSpec((blk, DIM), lambda i: (i, 0)),
            pl.BlockSpec((blk, DIM), lambda i: (i, 0)),
            pl.BlockSpec((DIM, 64), lambda i: (0, 0)),
            pl.BlockSpec((1, 64), lambda i: (0, 0)),
            pl.BlockSpec((1, 1), lambda i: (0, 0)),
        ],
        out_specs=[
            pl.BlockSpec((blk, 16), lambda i: (i, 0)),
            pl.BlockSpec((blk, DIM), lambda i: (i, 0)),
        ],
        out_shape=[_f32((NNODE, 16)), _f32((NNODE, DIM))],
    )(h, e_in, w1h, w2r, b2r)


def _tc_fuse_body(wsa_ref, za_ref, wsb_ref, zb_ref, f1_ref, fb1_ref, f2_ref,
                  fb2_ref, hi_ref, ha_ref, hb_ref):
    za = za_ref[:, 0:1]
    zb = zb_ref[:, 0:1]
    ha = wsa_ref[...] / (za + 1e-16)
    hb = wsb_ref[...] / (zb + 1e-16)
    ha_ref[...] = ha
    hb_ref[...] = hb
    cat = jnp.concatenate([ha, hb, ha * hb, jnp.abs(ha - hb)], axis=1)
    hid = jnp.maximum(jnp.dot(cat, f1_ref[...],
                              preferred_element_type=jnp.float32)
                      + fb1_ref[...], 0.0)
    hi_ref[...] = jnp.dot(hid, f2_ref[...],
                          preferred_element_type=jnp.float32) + fb2_ref[...]


def _tc_fuse(ws_a, z_a, ws_b, z_b, f1, fb1r, f2, fb2r):
    blk = 1000
    return pl.pallas_call(
        _tc_fuse_body,
        grid=(NSEG // blk,),
        in_specs=[
            pl.BlockSpec((blk, DIM), lambda i: (i, 0)),
            pl.BlockSpec((blk, 16), lambda i: (i, 0)),
            pl.BlockSpec((blk, DIM), lambda i: (i, 0)),
            pl.BlockSpec((blk, 16), lambda i: (i, 0)),
            pl.BlockSpec((4 * DIM, 2 * DIM), lambda i: (0, 0)),
            pl.BlockSpec((1, 2 * DIM), lambda i: (0, 0)),
            pl.BlockSpec((2 * DIM, DIM), lambda i: (0, 0)),
            pl.BlockSpec((1, DIM), lambda i: (0, 0)),
        ],
        out_specs=[
            pl.BlockSpec((blk, DIM), lambda i: (i, 0)),
            pl.BlockSpec((blk, DIM), lambda i: (i, 0)),
            pl.BlockSpec((blk, DIM), lambda i: (i, 0)),
        ],
        out_shape=[_f32((NSEG, DIM)), _f32((NSEG, DIM)), _f32((NSEG, DIM))],
    )(ws_a, z_a, ws_b, z_b, f1, fb1r, f2, fb2r)


def kernel(h_A, herb_idx_A, h_B, herb_idx_B, W1, b1, W2, b2, F1, fb1, F2, fb2):
    idx_a3 = herb_idx_A.reshape(NBLK, 1, RPB)
    idx_b3 = herb_idx_B.reshape(NBLK, 1, RPB)
    idx_ar = herb_idx_A.reshape(NRB, 1, RB)
    idx_br = herb_idx_B.reshape(NRB, 1, RB)

    s_a, c_a = _tc_segcnt(h_A, idx_ar)
    s_b, c_b = _tc_segcnt(h_B, idx_br)

    w1h = W1[:DIM]
    w1g = W1[DIM:]
    b1r = b1.reshape(1, 64)
    v_a = _tc_vtab(s_a, c_a, w1g, b1r)
    v_b = _tc_vtab(s_b, c_b, w1g, b1r)

    v_stack = jnp.stack([v_b, v_a])
    idx_stack = jnp.stack([herb_idx_A.reshape(NBLKE, NQ, RPB),
                           herb_idx_B.reshape(NBLKE, NQ, RPB)])
    e_exp = _sc_expand(v_stack, idx_stack)
    e_in_a = e_exp[0].reshape(NNODE, DIM)
    e_in_b = e_exp[1].reshape(NNODE, DIM)

    w2r = W2.reshape(1, 64)
    b2r = b2.reshape(1, 1)
    e_a, eh_a = _tc_score(h_A, e_in_a, w1h, w2r, b2r)
    e_b, eh_b = _tc_score(h_B, e_in_b, w1h, w2r, b2r)

    ws_a, z_a = _tc_segpay(eh_a, e_a, idx_ar)
    ws_b, z_b = _tc_segpay(eh_b, e_b, idx_br)

    h_int, h_agb, h_bga = _tc_fuse(ws_a, z_a, ws_b, z_b, F1,
                                   fb1.reshape(1, 2 * DIM), F2,
                                   fb2.reshape(1, DIM))
    return (h_int, h_agb, h_bga)
